# baseline (speedup 1.0000x reference)
import functools

import jax
import jax.numpy as jnp
from jax import lax
from jax.experimental import pallas as pl
from jax.experimental.pallas import tpu as pltpu
from jax.experimental.pallas import tpu_sc as plsc

_N = 10000
_D = 128
_NC = 2
_NS = 16
_NW = _NC * _NS
_CH = 120
_CHA = 64
_CHB = _CH - _CHA
_NPAD = 10112
_RPT = _NPAD // _NS
_LANES = 16


def _spmm_body(nchunk, x_hbm, src_hbm, dsta_hbm, dstb_hbm, val_hbm, out_hbm,
               src_v, dva_v, dvb_v, val_a, val_b, rows_a, rows_b, acc_sh,
               gsa, gsb, sAa, sAb, sBa, sBb, msa, msb):
    cid = lax.axis_index("c")
    sid = lax.axis_index("s")
    wid = cid * _NS + sid

    pltpu.sync_copy(src_hbm.at[wid], src_v)

    zeros = jnp.zeros((_LANES,), jnp.float32)

    def _zrow(r, _):
        for dd in range(_D // _LANES):
            rows_a[r, pl.ds(dd * _LANES, _LANES)] = zeros
        return 0

    lax.fori_loop(0, _CH, _zrow, 0)
    base_row = sid * _RPT
    done = 0
    while done < _RPT:
        step = min(_CH, _RPT - done)
        pltpu.sync_copy(rows_a.at[pl.ds(0, step)],
                        acc_sh.at[pl.ds(base_row + done, step)])
        done += step
    plsc.subcore_barrier()

    rows = (rows_a, rows_b)
    vals = (val_a, val_b)
    gs = (gsa, gsb)
    sA = (sAa, sAb)
    sB = (sBa, sBb)
    ms = (msa, msb)

    def issue_meta(j, b):
        pltpu.async_copy(dsta_hbm.at[wid, j], dva_v.at[b], ms[b])
        pltpu.async_copy(dstb_hbm.at[wid, j], dvb_v.at[b], ms[b])
        pltpu.async_copy(val_hbm.at[wid, j], vals[b], ms[b])

    def wait_meta(j, b):
        pltpu.make_async_copy(dsta_hbm.at[wid, j], dva_v.at[b], ms[b]).wait()
        pltpu.make_async_copy(dstb_hbm.at[wid, j], dvb_v.at[b], ms[b]).wait()
        pltpu.make_async_copy(val_hbm.at[wid, j], vals[b], ms[b]).wait()

    def issue_gather(j, b):
        pltpu.async_copy(x_hbm.at[src_v.at[j]], rows[b], gs[b])

    def wait_gather(j, b):
        pltpu.make_async_copy(x_hbm.at[src_v.at[j]], rows[b], gs[b]).wait()

    def issue_scatter_a(b):
        pltpu.async_copy(rows[b].at[pl.ds(0, _CHA)],
                         acc_sh.at[dva_v.at[b]], sA[b], add=True)

    def wait_scatter_a(b):
        pltpu.make_async_copy(rows[b].at[pl.ds(0, _CHA)],
                              acc_sh.at[dva_v.at[b]], sA[b]).wait()

    def issue_scatter_b(b):
        pltpu.async_copy(rows[b].at[pl.ds(_CHA, _CHB)],
                         acc_sh.at[dvb_v.at[b]], sB[b], add=True)

    def wait_scatter_b(b):
        pltpu.make_async_copy(rows[b].at[pl.ds(_CHA, _CHB)],
                              acc_sh.at[dvb_v.at[b]], sB[b]).wait()

    def multiply(b, lo, hi):
        r = rows[b]
        v = vals[b]

        @plsc.parallel_loop(lo, hi, 1, unroll=4)
        def _(e):
            vv = plsc.load_gather(v, [jnp.full((_LANES,), e, jnp.int32)])
            for dd in range(_D // _LANES):
                sl = pl.ds(dd * _LANES, _LANES)
                r[e, sl] = r[e, sl] * vv

    def process(b):
        multiply(b, 0, _CHA)
        issue_scatter_a(b)
        multiply(b, _CHA, _CH)
        wait_scatter_a(b)
        issue_scatter_b(b)

    issue_meta(0, 0)
    issue_gather(0, 0)
    issue_meta(1, 1)
    issue_gather(1, 1)
    wait_gather(0, 0)
    wait_meta(0, 0)
    process(0)

    def _pair(k, _):
        for b in (1, 0):
            j = 2 * k + (1 if b == 1 else 2)
            wait_scatter_b(1 - b)
            issue_meta(j + 1, 1 - b)
            issue_gather(j + 1, 1 - b)
            wait_gather(j, b)
            wait_meta(j, b)
            process(b)
        return 0

    lax.fori_loop(0, (nchunk - 2) // 2, _pair, 0)

    jl = nchunk - 1
    wait_scatter_b(0)
    wait_gather(jl, 1)
    wait_meta(jl, 1)
    process(1)
    wait_scatter_b(1)

    plsc.subcore_barrier()

    pltpu.sync_copy(acc_sh.at[pl.ds(base_row, _RPT)],
                    out_hbm.at[cid, pl.ds(base_row, _RPT)])


@functools.partial(jax.jit, static_argnames=("nchunk",))
def _spmm_sc(x, src, dsta, dstb, val, nchunk):
    mesh = plsc.VectorSubcoreMesh(core_axis_name="c", subcore_axis_name="s",
                                  num_cores=_NC, num_subcores=_NS)
    kfn = pl.kernel(
        functools.partial(_spmm_body, nchunk),
        out_type=jax.ShapeDtypeStruct((_NC, _NPAD, _D), jnp.float32),
        mesh=mesh,
        scratch_types=[
            pltpu.VMEM((nchunk, _CH), jnp.int32),
            pltpu.VMEM((2, _CHA), jnp.int32),
            pltpu.VMEM((2, _CHB), jnp.int32),
            pltpu.VMEM((_CH,), jnp.float32),
            pltpu.VMEM((_CH,), jnp.float32),
            pltpu.VMEM((_CH, _D), jnp.float32),
            pltpu.VMEM((_CH, _D), jnp.float32),
            pltpu.VMEM_SHARED((_NPAD, _D), jnp.float32),
            pltpu.SemaphoreType.DMA,
            pltpu.SemaphoreType.DMA,
            pltpu.SemaphoreType.DMA,
            pltpu.SemaphoreType.DMA,
            pltpu.SemaphoreType.DMA,
            pltpu.SemaphoreType.DMA,
            pltpu.SemaphoreType.DMA,
            pltpu.SemaphoreType.DMA,
        ],
        compiler_params=pltpu.CompilerParams(needs_layout_passes=False),
    )
    return kfn(x, src, dsta, dstb, val)


_BLK = 1000


def _sg_body(emb_ref, w0_ref, b0_ref, w1_ref, b1_ref, ui_ref, uu_ref):
    x = emb_ref[...]
    ui_ref[...] = x * jax.nn.sigmoid(
        jnp.dot(x, w0_ref[...], preferred_element_type=jnp.float32)
        + b0_ref[...])
    uu_ref[...] = x * jax.nn.sigmoid(
        jnp.dot(x, w1_ref[...], preferred_element_type=jnp.float32)
        + b1_ref[...])


def _selfgate(emb, w0, b0, w1, b1):
    n = emb.shape[0]
    grid = (n // _BLK,)
    row = pl.BlockSpec((_BLK, _D), lambda i: (i, 0))
    mat = pl.BlockSpec((_D, _D), lambda i: (0, 0))
    vec = pl.BlockSpec((1, _D), lambda i: (0, 0))
    return pl.pallas_call(
        _sg_body,
        grid=grid,
        in_specs=[row, mat, vec, mat, vec],
        out_specs=[row, row],
        out_shape=[jax.ShapeDtypeStruct((n, _D), jnp.float32)] * 2,
    )(emb, w0, b0, w1, b1)


def _comb_body(p_ref, o_ref):
    o_ref[...] = p_ref[0] + p_ref[1]


def _combine(p):
    n = p.shape[1]
    grid = (n // _BLK,)
    return pl.pallas_call(
        _comb_body,
        grid=grid,
        in_specs=[pl.BlockSpec((_NC, _BLK, _D), lambda i: (0, i, 0))],
        out_specs=pl.BlockSpec((_BLK, _D), lambda i: (i, 0)),
        out_shape=jax.ShapeDtypeStruct((n, _D), jnp.float32),
    )(p)


def _fin_body(ui_ref, xi1_ref, pi2_ref, uu_ref, xu1_ref, pu2_ref,
              att_ref, attm_ref, fw1_ref, fb1_ref, fw2_ref, out_ref):
    third = jnp.float32(1.0 / 3.0)
    ei = (ui_ref[...] + xi1_ref[...] + pi2_ref[0] + pi2_ref[1]) * third
    eu = (uu_ref[...] + xu1_ref[...] + pu2_ref[0] + pu2_ref[1]) * third

    t = jnp.dot(ei - eu, attm_ref[...], preferred_element_type=jnp.float32)
    dw = jnp.sum(t * att_ref[...], axis=1)
    s0 = jax.nn.sigmoid(dw)
    mixed = s0[:, None] * ei + (1.0 - s0)[:, None] * eu

    h0 = jnp.tanh(
        lax.dot_general(mixed, fw1_ref[...], (((1,), (1,)), ((), ())),
                        preferred_element_type=jnp.float32) + fb1_ref[...])
    h1 = jnp.tanh(
        lax.dot_general(eu, fw1_ref[...], (((1,), (1,)), ((), ())),
                        preferred_element_type=jnp.float32) + fb1_ref[...])
    g0 = jnp.sum(h0 * fw2_ref[...], axis=1)
    g1 = jnp.sum(h1 * fw2_ref[...], axis=1)
    sf = jax.nn.sigmoid(g0 - g1)
    out_ref[...] = sf[:, None] * mixed + (1.0 - sf)[:, None] * eu


def _final(ui, xi1, pi2, uu, xu1, pu2, att, att_m, fw1, fb1, fw2):
    n = ui.shape[0]
    grid = (n // _BLK,)
    row = pl.BlockSpec((_BLK, _D), lambda i: (i, 0))
    par = pl.BlockSpec((_NC, _BLK, _D), lambda i: (0, i, 0))
    mat = pl.BlockSpec((_D, _D), lambda i: (0, 0))
    vec = pl.BlockSpec((1, _D), lambda i: (0, 0))
    return pl.pallas_call(
        _fin_body,
        grid=grid,
        in_specs=[row, row, par, row, row, par, vec, mat, mat, vec, vec],
        out_specs=row,
        out_shape=jax.ShapeDtypeStruct((n, _D), jnp.float32),
    )(ui, xi1, pi2, uu, xu1, pu2, att, att_m, fw1, fb1, fw2)


def _edges_tiled(edge_index, edge_val):
    e = edge_index.shape[1]
    per_tile = -(-e // _NW)
    nchunk = max(-(-per_tile // _CH), 8)
    nchunk = ((nchunk + 7) // 8) * 8
    e_pad = _NW * nchunk * _CH
    idx = edge_index.astype(jnp.int32)
    src = idx[1]
    dst = idx[0]
    val = edge_val.astype(jnp.float32)
    if e_pad != e:
        pad = e_pad - e
        src = jnp.pad(src, (0, pad))
        dst = jnp.pad(dst, (0, pad))
        val = jnp.pad(val, (0, pad))
    shape = (_NW, nchunk, _CH)
    dst_t = dst.reshape(shape)
    return (src.reshape(shape), dst_t[:, :, :_CHA].copy(),
            dst_t[:, :, _CHA:].copy(), val.reshape(shape), nchunk)


def kernel(emb_table, W0, b0, W1, b1, att, att_m, fuse_W1, fuse_b1,
           fuse_W2, fuse_b2, item_edge_index, item_edge_val,
           user_edge_index, user_edge_val):
    isrc, idsta, idstb, ival, inch = _edges_tiled(item_edge_index,
                                                  item_edge_val)
    usrc, udsta, udstb, uval, unch = _edges_tiled(user_edge_index,
                                                  user_edge_val)

    ui, uu = _selfgate(emb_table, W0, b0, W1, b1)

    pi1 = _spmm_sc(ui, isrc, idsta, idstb, ival, inch)
    xi1 = _combine(pi1)
    pi2 = _spmm_sc(xi1, isrc, idsta, idstb, ival, inch)

    pu1 = _spmm_sc(uu, usrc, udsta, udstb, uval, unch)
    xu1 = _combine(pu1)
    pu2 = _spmm_sc(xu1, usrc, udsta, udstb, uval, unch)

    return _final(ui, xi1, pi2, uu, xu1, pu2, att, att_m,
                  fuse_W1, fuse_b1.reshape(1, _D), fuse_W2)


# speedup vs baseline: 6.7849x; 6.7849x over previous
import functools

import jax
import jax.numpy as jnp
from jax import lax
from jax.experimental import pallas as pl
from jax.experimental.pallas import tpu as pltpu
from jax.experimental.pallas import tpu_sc as plsc

_N = 10000
_D = 128
_NC = 2
_NS = 16
_NW = _NC * _NS
_CH = 125
_CHP = 128
_NPAD = 10112
_RPT = _NPAD // _NS
_LANES = 16


def _spmm_body(nchunk, x_hbm, src_hbm, dst_hbm, val_hbm, out_hbm,
               src_v, dv_v, val_a, val_b, rows_a, rows_b, acc_sh,
               gsa, gsb, ssa, ssb, msa, msb):
    cid = lax.axis_index("c")
    sid = lax.axis_index("s")
    wid = cid * _NS + sid

    pltpu.sync_copy(src_hbm.at[wid], src_v)

    zeros = jnp.zeros((_LANES,), jnp.float32)

    def _zrow(r, _):
        for dd in range(_D // _LANES):
            rows_a[r, pl.ds(dd * _LANES, _LANES)] = zeros
        return 0

    lax.fori_loop(0, 120, _zrow, 0)
    base_row = sid * _RPT
    done = 0
    while done < _RPT:
        step = min(120, _RPT - done)
        pltpu.sync_copy(rows_a.at[pl.ds(0, step)],
                        acc_sh.at[pl.ds(base_row + done, step)])
        done += step
    plsc.subcore_barrier()

    rows = (rows_a, rows_b)
    vals = (val_a, val_b)
    gs = (gsa, gsb)
    ss = (ssa, ssb)
    ms = (msa, msb)

    def issue_meta(j, b):
        pltpu.async_copy(dst_hbm.at[wid, j], dv_v.at[b], ms[b])
        pltpu.async_copy(val_hbm.at[wid, j], vals[b], ms[b])

    def wait_meta(j, b):
        pltpu.make_async_copy(dst_hbm.at[wid, j], dv_v.at[b], ms[b]).wait()
        pltpu.make_async_copy(val_hbm.at[wid, j], vals[b], ms[b]).wait()

    def issue_gather(j, b):
        pltpu.async_copy(x_hbm.at[src_v.at[j]], rows[b], gs[b])

    def wait_gather(j, b):
        pltpu.make_async_copy(x_hbm.at[src_v.at[j]], rows[b], gs[b]).wait()

    def issue_scatter(b):
        pltpu.async_copy(rows[b], acc_sh.at[dv_v.at[b]], ss[b], add=True)

    def wait_scatter(b):
        pltpu.make_async_copy(rows[b], acc_sh.at[dv_v.at[b]], ss[b]).wait()

    def multiply(b):
        r = rows[b]
        v = vals[b]

        @plsc.parallel_loop(0, _CH, 1, unroll=5)
        def _(e):
            vv = plsc.load_gather(v, [jnp.full((_LANES,), e, jnp.int32)])
            for dd in range(_D // _LANES):
                sl = pl.ds(dd * _LANES, _LANES)
                r[e, sl] = r[e, sl] * vv

    issue_meta(0, 0)
    issue_gather(0, 0)
    issue_meta(1, 1)
    issue_gather(1, 1)
    wait_gather(0, 0)
    wait_meta(0, 0)
    multiply(0)
    issue_scatter(0)

    def _pair(k, _):
        for b in (1, 0):
            j = 2 * k + (1 if b == 1 else 2)
            wait_scatter(1 - b)
            issue_meta(j + 1, 1 - b)
            issue_gather(j + 1, 1 - b)
            wait_gather(j, b)
            wait_meta(j, b)
            multiply(b)
            issue_scatter(b)
        return 0

    lax.fori_loop(0, (nchunk - 2) // 2, _pair, 0)

    jl = nchunk - 1
    wait_scatter(0)
    wait_gather(jl, 1)
    wait_meta(jl, 1)
    multiply(1)
    issue_scatter(1)
    wait_scatter(1)

    plsc.subcore_barrier()

    pltpu.sync_copy(acc_sh.at[pl.ds(base_row, _RPT)],
                    out_hbm.at[cid, pl.ds(base_row, _RPT)])


@functools.partial(jax.jit, static_argnames=("nchunk",))
def _spmm_sc(x, src, dst, val, nchunk):
    mesh = plsc.VectorSubcoreMesh(core_axis_name="c", subcore_axis_name="s",
                                  num_cores=_NC, num_subcores=_NS)
    kfn = pl.kernel(
        functools.partial(_spmm_body, nchunk),
        out_type=jax.ShapeDtypeStruct((_NC, _NPAD, _D), jnp.float32),
        mesh=mesh,
        scratch_types=[
            pltpu.VMEM((nchunk, _CH), jnp.int32),
            pltpu.VMEM((2, _CH), jnp.int32),
            pltpu.VMEM((_CH,), jnp.float32),
            pltpu.VMEM((_CH,), jnp.float32),
            pltpu.VMEM((_CH, _D), jnp.float32),
            pltpu.VMEM((_CH, _D), jnp.float32),
            pltpu.VMEM_SHARED((_NPAD, _D), jnp.float32),
            pltpu.SemaphoreType.DMA,
            pltpu.SemaphoreType.DMA,
            pltpu.SemaphoreType.DMA,
            pltpu.SemaphoreType.DMA,
            pltpu.SemaphoreType.DMA,
            pltpu.SemaphoreType.DMA,
        ],
        compiler_params=pltpu.CompilerParams(needs_layout_passes=False),
    )
    return kfn(x, src, dst, val)


_BLK = 1000


def _sg_body(emb_ref, w0_ref, b0_ref, w1_ref, b1_ref, ui_ref, uu_ref):
    x = emb_ref[...]
    ui_ref[...] = x * jax.nn.sigmoid(
        jnp.dot(x, w0_ref[...], preferred_element_type=jnp.float32)
        + b0_ref[...])
    uu_ref[...] = x * jax.nn.sigmoid(
        jnp.dot(x, w1_ref[...], preferred_element_type=jnp.float32)
        + b1_ref[...])


def _selfgate(emb, w0, b0, w1, b1):
    n = emb.shape[0]
    grid = (n // _BLK,)
    row = pl.BlockSpec((_BLK, _D), lambda i: (i, 0))
    mat = pl.BlockSpec((_D, _D), lambda i: (0, 0))
    vec = pl.BlockSpec((1, _D), lambda i: (0, 0))
    return pl.pallas_call(
        _sg_body,
        grid=grid,
        in_specs=[row, mat, vec, mat, vec],
        out_specs=[row, row],
        out_shape=[jax.ShapeDtypeStruct((n, _D), jnp.float32)] * 2,
    )(emb, w0, b0, w1, b1)


def _comb_body(p_ref, o_ref):
    o_ref[...] = p_ref[0] + p_ref[1]


def _combine(p):
    n = p.shape[1]
    grid = (n // _BLK,)
    return pl.pallas_call(
        _comb_body,
        grid=grid,
        in_specs=[pl.BlockSpec((_NC, _BLK, _D), lambda i: (0, i, 0))],
        out_specs=pl.BlockSpec((_BLK, _D), lambda i: (i, 0)),
        out_shape=jax.ShapeDtypeStruct((n, _D), jnp.float32),
    )(p)


def _fin_body(ui_ref, xi1_ref, pi2_ref, uu_ref, xu1_ref, pu2_ref,
              att_ref, attm_ref, fw1_ref, fb1_ref, fw2_ref, out_ref):
    third = jnp.float32(1.0 / 3.0)
    ei = (ui_ref[...] + xi1_ref[...] + pi2_ref[0] + pi2_ref[1]) * third
    eu = (uu_ref[...] + xu1_ref[...] + pu2_ref[0] + pu2_ref[1]) * third

    t = jnp.dot(ei - eu, attm_ref[...], preferred_element_type=jnp.float32)
    dw = jnp.sum(t * att_ref[...], axis=1)
    s0 = jax.nn.sigmoid(dw)
    mixed = s0[:, None] * ei + (1.0 - s0)[:, None] * eu

    h0 = jnp.tanh(
        lax.dot_general(mixed, fw1_ref[...], (((1,), (1,)), ((), ())),
                        preferred_element_type=jnp.float32) + fb1_ref[...])
    h1 = jnp.tanh(
        lax.dot_general(eu, fw1_ref[...], (((1,), (1,)), ((), ())),
                        preferred_element_type=jnp.float32) + fb1_ref[...])
    g0 = jnp.sum(h0 * fw2_ref[...], axis=1)
    g1 = jnp.sum(h1 * fw2_ref[...], axis=1)
    sf = jax.nn.sigmoid(g0 - g1)
    out_ref[...] = sf[:, None] * mixed + (1.0 - sf)[:, None] * eu


def _final(ui, xi1, pi2, uu, xu1, pu2, att, att_m, fw1, fb1, fw2):
    n = ui.shape[0]
    grid = (n // _BLK,)
    row = pl.BlockSpec((_BLK, _D), lambda i: (i, 0))
    par = pl.BlockSpec((_NC, _BLK, _D), lambda i: (0, i, 0))
    mat = pl.BlockSpec((_D, _D), lambda i: (0, 0))
    vec = pl.BlockSpec((1, _D), lambda i: (0, 0))
    return pl.pallas_call(
        _fin_body,
        grid=grid,
        in_specs=[row, row, par, row, row, par, vec, mat, mat, vec, vec],
        out_specs=row,
        out_shape=jax.ShapeDtypeStruct((n, _D), jnp.float32),
    )(ui, xi1, pi2, uu, xu1, pu2, att, att_m, fw1, fb1, fw2)


def _edges_tiled(edge_index, edge_val):
    e = edge_index.shape[1]
    quantum = 2 * _NW * _CH
    e_pad = ((e + quantum - 1) // quantum) * quantum
    idx = edge_index.astype(jnp.int32)
    src = idx[1]
    dst = idx[0]
    val = edge_val.astype(jnp.float32)
    if e_pad != e:
        pad = e_pad - e
        src = jnp.pad(src, (0, pad))
        dst = jnp.pad(dst, (0, pad))
        val = jnp.pad(val, (0, pad))
    nchunk = e_pad // (_NW * _CH)
    shape = (_NW, nchunk, _CH)
    return (src.reshape(shape), dst.reshape(shape), val.reshape(shape),
            nchunk)


def kernel(emb_table, W0, b0, W1, b1, att, att_m, fuse_W1, fuse_b1,
           fuse_W2, fuse_b2, item_edge_index, item_edge_val,
           user_edge_index, user_edge_val):
    isrc, idst, ival, inch = _edges_tiled(item_edge_index, item_edge_val)
    usrc, udst, uval, unch = _edges_tiled(user_edge_index, user_edge_val)

    ui, uu = _selfgate(emb_table, W0, b0, W1, b1)

    pi1 = _spmm_sc(ui, isrc, idst, ival, inch)
    xi1 = _combine(pi1)
    pi2 = _spmm_sc(xi1, isrc, idst, ival, inch)

    pu1 = _spmm_sc(uu, usrc, udst, uval, unch)
    xu1 = _combine(pu1)
    pu2 = _spmm_sc(xu1, usrc, udst, uval, unch)

    return _final(ui, xi1, pi2, uu, xu1, pu2, att, att_m,
                  fuse_W1, fuse_b1.reshape(1, _D), fuse_W2)
